# baseline (speedup 1.0000x reference)
import functools

import jax
import jax.numpy as jnp
import numpy as np
from jax import lax
from jax.experimental import pallas as pl
from jax.experimental.pallas import tpu as pltpu
from jax.experimental.pallas import tpu_sc as plsc

NC = 2
NS = 16
NW = NC * NS
LANE = 16

_SPLAT_DNUMS = lax.GatherDimensionNumbers(
    offset_dims=(), collapsed_slice_dims=(0,), start_index_map=(0,))


def _splat_lane(vec, lane):
    idx = jnp.full((LANE, 1), lane, jnp.int32)
    return lax.gather(vec, idx, _SPLAT_DNUMS, (1,),
                      mode=lax.GatherScatterMode.PROMISE_IN_BOUNDS)


def _pool_sc(tokens, emb32, boost, L, LP, B):
    V, DW = emb32.shape
    D = 2 * DW
    rows_per_w = B // NW
    CH = 4
    nch = rows_per_w // CH
    TOK = rows_per_w * LP
    inv_l = 1.0 / L
    mesh = plsc.VectorSubcoreMesh(core_axis_name="c", subcore_axis_name="s")

    @functools.partial(
        pl.kernel,
        out_type=jax.ShapeDtypeStruct((B, D), jnp.float32),
        mesh=mesh,
        compiler_params=pltpu.CompilerParams(use_tc_tiling_on_sc=False),
        scratch_types=[
            pltpu.VMEM((TOK,), jnp.int32),
            pltpu.VMEM((rows_per_w * 128,), jnp.float32),
            pltpu.VMEM((CH * L, DW), jnp.int32),
            pltpu.VMEM((CH * L, DW), jnp.int32),
            pltpu.VMEM((rows_per_w, D), jnp.float32),
            pltpu.SemaphoreType.DMA,
            pltpu.SemaphoreType.DMA,
            pltpu.SemaphoreType.DMA,
            pltpu.SemaphoreType.DMA,
        ],
    )
    def k(t_hbm, e_hbm, b_hbm, out_hbm,
          idx_t, bst_t, rows_a, rows_b, acc_v,
          sem_ra, sem_rb, sem_ba, sem_bb):
        wid = lax.axis_index("s") * NC + lax.axis_index("c")
        base = wid * rows_per_w
        pltpu.sync_copy(t_hbm.at[pl.ds(base * LP, TOK)], idx_t)
        bufs = ((rows_a, sem_ra, sem_ba), (rows_b, sem_rb, sem_bb))

        def issue(ch, rbuf, sr, sb):
            for r in range(CH):
                isl = idx_t.at[pl.ds((ch * CH + r) * LP, L)]
                pltpu.async_copy(e_hbm.at[isl],
                                 rbuf.at[pl.ds(r * L, L)], sr)
                pltpu.async_copy(b_hbm.at[isl],
                                 bst_t.at[pl.ds((ch * CH + r) * 128, L)],
                                 sb)

        def wait(ch, rbuf, sr, sb):
            pltpu.make_async_copy(
                b_hbm.at[idx_t.at[pl.ds(ch * CH * LP, CH * L)]],
                bst_t.at[pl.ds(ch * CH * 128, CH * L)], sb).wait()
            pltpu.make_async_copy(
                e_hbm.at[idx_t.at[pl.ds(ch * CH * LP, CH * L)]],
                rbuf, sr).wait()

        def compute(ch, rbuf):
            def row_body(r, carry):
                grow = ch * CH + r
                wvecs = [bst_t[pl.ds(grow * 128 + g * LANE, LANE)] * inv_l
                         for g in range((L + LANE - 1) // LANE)]
                accs = [jnp.zeros((LANE,), jnp.float32)
                        for _ in range(D // LANE)]
                for l in range(L):
                    wsplat = _splat_lane(wvecs[l // LANE], l % LANE)
                    for c in range(DW // LANE):
                        w32 = rbuf[r * L + l, pl.ds(c * LANE, LANE)]
                        ev = lax.bitcast_convert_type(w32 << 16, jnp.float32)
                        od = lax.bitcast_convert_type(
                            w32 & jnp.int32(-65536), jnp.float32)
                        accs[2 * c] = accs[2 * c] + ev * wsplat
                        accs[2 * c + 1] = accs[2 * c + 1] + od * wsplat
                for c in range(D // LANE):
                    acc_v[grow, pl.ds(c * LANE, LANE)] = accs[c]
                return carry

            lax.fori_loop(0, CH, row_body, 0)

        issue(0, *bufs[0])

        def body2(i, carry):
            ch0 = 2 * i
            issue(ch0 + 1, *bufs[1])
            wait(ch0, *bufs[0])
            compute(ch0, bufs[0][0])

            @pl.when(ch0 + 2 < nch)
            def _():
                issue(ch0 + 2, *bufs[0])

            wait(ch0 + 1, *bufs[1])
            compute(ch0 + 1, bufs[1][0])
            return carry

        lax.fori_loop(0, nch // 2, body2, 0)
        pltpu.sync_copy(acc_v, out_hbm.at[pl.ds(base, rows_per_w)])

    return k(tokens, emb32, boost)


def _layer_norm(x, g, b):
    mu = jnp.mean(x, axis=-1, keepdims=True)
    var = jnp.mean((x - mu) ** 2, axis=-1, keepdims=True)
    return (x - mu) * lax.rsqrt(var + 1e-5) * g + b


def _mlp_tc(x, W1, b1, g1, be1, W2, b2, g2, be2, oW, ob):
    B, D = x.shape
    H1 = W1.shape[-1]
    H2 = W2.shape[-1]
    OUT = oW.shape[-1]
    BS = 512

    def bdot(a, b):
        return jnp.dot(a.astype(jnp.bfloat16), b.astype(jnp.bfloat16),
                       preferred_element_type=jnp.float32)

    def body(x_ref, w1_ref, b1_ref, g1_ref, be1_ref, w2_ref, b2_ref, g2_ref,
             be2_ref, ow_ref, ob_ref, o_ref):
        h = bdot(x_ref[...], w1_ref[...])
        h = jnp.maximum(h + b1_ref[...], 0.0)
        h = _layer_norm(h, g1_ref[...], be1_ref[...])
        h = bdot(h, w2_ref[...])
        h = jnp.maximum(h + b2_ref[...], 0.0)
        h = _layer_norm(h, g2_ref[...], be2_ref[...])
        o_ref[...] = bdot(h, ow_ref[...]) + ob_ref[...]

    def wspec(shape):
        return pl.BlockSpec(shape, lambda i: (0,) * len(shape))

    return pl.pallas_call(
        body,
        grid=(B // BS,),
        in_specs=[
            pl.BlockSpec((BS, D), lambda i: (i, 0)),
            wspec((D, H1)), wspec((1, H1)), wspec((1, H1)), wspec((1, H1)),
            wspec((H1, H2)), wspec((1, H2)), wspec((1, H2)), wspec((1, H2)),
            wspec((H2, OUT)), wspec((1, OUT)),
        ],
        out_specs=pl.BlockSpec((BS, OUT), lambda i: (i, 0)),
        out_shape=jax.ShapeDtypeStruct((B, OUT), jnp.float32),
    )(x, W1, b1, g1, be1, W2, b2, g2, be2, oW, ob)


def kernel(query_tokens, doc_tokens,
           q_emb_table, q_boost_table, q_W1, q_b1, q_g1, q_be1,
           q_W2, q_b2, q_g2, q_be2, q_oW, q_ob,
           d_emb_table, d_boost_table, d_W1, d_b1, d_g1, d_be1,
           d_W2, d_b2, d_g2, d_be2, d_oW, d_ob):
    B, L = query_tokens.shape
    V, D = q_emb_table.shape
    LP = ((L + 7) // 8) * 8
    qt = jnp.pad(query_tokens, ((0, 0), (0, LP - L))).reshape(-1)
    dt = jnp.pad(doc_tokens, ((0, 0), (0, LP - L))).reshape(-1)

    def as_words(emb):
        return lax.bitcast_convert_type(
            emb.astype(jnp.bfloat16).reshape(V, D // 2, 2), jnp.int32)

    perm = np.concatenate(
        [np.concatenate([np.arange(32 * c, 32 * c + 32, 2),
                         np.arange(32 * c + 1, 32 * c + 32, 2)])
         for c in range(D // 32)])
    pq = _pool_sc(qt, as_words(q_emb_table), q_boost_table, L, LP, B)
    pd = _pool_sc(dt, as_words(d_emb_table), d_boost_table, L, LP, B)
    oq = _mlp_tc(pq, q_W1[perm], q_b1[None], q_g1[None], q_be1[None],
                 q_W2, q_b2[None], q_g2[None], q_be2[None], q_oW, q_ob[None])
    od = _mlp_tc(pd, d_W1[perm], d_b1[None], d_g1[None], d_be1[None],
                 d_W2, d_b2[None], d_g2[None], d_be2[None], d_oW, d_ob[None])
    return jnp.stack([oq, od])


# speedup vs baseline: 5.9064x; 5.9064x over previous
import functools

import jax
import jax.numpy as jnp
import numpy as np
from jax import lax
from jax.experimental import pallas as pl
from jax.experimental.pallas import tpu as pltpu
from jax.experimental.pallas import tpu_sc as plsc

NC = 2
NS = 16
NW = NC * NS
LANE = 16

_SPLAT_DNUMS = lax.GatherDimensionNumbers(
    offset_dims=(), collapsed_slice_dims=(0,), start_index_map=(0,))


def _splat_lane(vec, lane):
    idx = jnp.full((LANE, 1), lane, jnp.int32)
    return lax.gather(vec, idx, _SPLAT_DNUMS, (1,),
                      mode=lax.GatherScatterMode.PROMISE_IN_BOUNDS)


def _pool_sc(tokens, emb, boost, L, LP, B):
    V, D = emb.shape
    rows_per_w = B // NW
    CH = 4
    nch = rows_per_w // CH
    TOK = rows_per_w * LP
    inv_l = 1.0 / L
    mesh = plsc.VectorSubcoreMesh(core_axis_name="c", subcore_axis_name="s")

    @functools.partial(
        pl.kernel,
        out_type=jax.ShapeDtypeStruct((B, D), jnp.float32),
        mesh=mesh,
        scratch_types=[
            pltpu.VMEM((TOK,), jnp.int32),
            pltpu.VMEM((rows_per_w * 128,), jnp.float32),
            pltpu.VMEM((CH * L, D), jnp.float32),
            pltpu.VMEM((CH * L, D), jnp.float32),
            pltpu.VMEM((rows_per_w, D), jnp.float32),
            pltpu.SemaphoreType.DMA,
            pltpu.SemaphoreType.DMA,
            pltpu.SemaphoreType.DMA,
            pltpu.SemaphoreType.DMA,
        ],
    )
    def k(t_hbm, e_hbm, b_hbm, out_hbm,
          idx_t, bst_t, rows_a, rows_b, acc_v,
          sem_ra, sem_rb, sem_ba, sem_bb):
        wid = lax.axis_index("s") * NC + lax.axis_index("c")
        base = wid * rows_per_w
        pltpu.sync_copy(t_hbm.at[pl.ds(base * LP, TOK)], idx_t)
        bufs = ((rows_a, sem_ra, sem_ba), (rows_b, sem_rb, sem_bb))

        def issue(ch, rbuf, sr, sb):
            for r in range(CH):
                isl = idx_t.at[pl.ds((ch * CH + r) * LP, L)]
                pltpu.async_copy(e_hbm.at[isl],
                                 rbuf.at[pl.ds(r * L, L)], sr)
                pltpu.async_copy(b_hbm.at[isl],
                                 bst_t.at[pl.ds((ch * CH + r) * 128, L)],
                                 sb)

        def wait(ch, rbuf, sr, sb):
            pltpu.make_async_copy(
                b_hbm.at[idx_t.at[pl.ds(ch * CH * LP, CH * L)]],
                bst_t.at[pl.ds(ch * CH * 128, CH * L)], sb).wait()
            pltpu.make_async_copy(
                e_hbm.at[idx_t.at[pl.ds(ch * CH * LP, CH * L)]],
                rbuf, sr).wait()

        def compute(ch, rbuf):
            def row_body(r, carry):
                grow = ch * CH + r
                wvecs = [bst_t[pl.ds(grow * 128 + g * LANE, LANE)] * inv_l
                         for g in range((L + LANE - 1) // LANE)]
                accs = [jnp.zeros((LANE,), jnp.float32)
                        for _ in range(D // LANE)]
                for l in range(L):
                    wsplat = _splat_lane(wvecs[l // LANE], l % LANE)
                    for c in range(D // LANE):
                        accs[c] = (accs[c]
                                   + rbuf[r * L + l, pl.ds(c * LANE, LANE)]
                                   * wsplat)
                for c in range(D // LANE):
                    acc_v[grow, pl.ds(c * LANE, LANE)] = accs[c]
                return carry

            lax.fori_loop(0, CH, row_body, 0)

        issue(0, *bufs[0])

        def body2(i, carry):
            ch0 = 2 * i
            issue(ch0 + 1, *bufs[1])
            wait(ch0, *bufs[0])
            compute(ch0, bufs[0][0])

            @pl.when(ch0 + 2 < nch)
            def _():
                issue(ch0 + 2, *bufs[0])

            wait(ch0 + 1, *bufs[1])
            compute(ch0 + 1, bufs[1][0])
            return carry

        lax.fori_loop(0, nch // 2, body2, 0)
        pltpu.sync_copy(acc_v, out_hbm.at[pl.ds(base, rows_per_w)])

    return k(tokens, emb, boost)


def _layer_norm(x, g, b):
    mu = jnp.mean(x, axis=-1, keepdims=True)
    var = jnp.mean((x - mu) ** 2, axis=-1, keepdims=True)
    return (x - mu) * lax.rsqrt(var + 1e-5) * g + b


def _mlp_tc(x, W1, b1, g1, be1, W2, b2, g2, be2, oW, ob):
    B, D = x.shape
    H1 = W1.shape[-1]
    H2 = W2.shape[-1]
    OUT = oW.shape[-1]
    BS = 512

    def bdot(a, b):
        return jnp.dot(a.astype(jnp.bfloat16), b.astype(jnp.bfloat16),
                       preferred_element_type=jnp.float32)

    def body(x_ref, w1_ref, b1_ref, g1_ref, be1_ref, w2_ref, b2_ref, g2_ref,
             be2_ref, ow_ref, ob_ref, o_ref):
        h = bdot(x_ref[...], w1_ref[...])
        h = jnp.maximum(h + b1_ref[...], 0.0)
        h = _layer_norm(h, g1_ref[...], be1_ref[...])
        h = bdot(h, w2_ref[...])
        h = jnp.maximum(h + b2_ref[...], 0.0)
        h = _layer_norm(h, g2_ref[...], be2_ref[...])
        o_ref[...] = bdot(h, ow_ref[...]) + ob_ref[...]

    def wspec(shape):
        return pl.BlockSpec(shape, lambda i: (0,) * len(shape))

    return pl.pallas_call(
        body,
        grid=(B // BS,),
        in_specs=[
            pl.BlockSpec((BS, D), lambda i: (i, 0)),
            wspec((D, H1)), wspec((1, H1)), wspec((1, H1)), wspec((1, H1)),
            wspec((H1, H2)), wspec((1, H2)), wspec((1, H2)), wspec((1, H2)),
            wspec((H2, OUT)), wspec((1, OUT)),
        ],
        out_specs=pl.BlockSpec((BS, OUT), lambda i: (i, 0)),
        out_shape=jax.ShapeDtypeStruct((B, OUT), jnp.float32),
    )(x, W1, b1, g1, be1, W2, b2, g2, be2, oW, ob)


def kernel(query_tokens, doc_tokens,
           q_emb_table, q_boost_table, q_W1, q_b1, q_g1, q_be1,
           q_W2, q_b2, q_g2, q_be2, q_oW, q_ob,
           d_emb_table, d_boost_table, d_W1, d_b1, d_g1, d_be1,
           d_W2, d_b2, d_g2, d_be2, d_oW, d_ob):
    B, L = query_tokens.shape
    LP = ((L + 7) // 8) * 8
    qt = jnp.pad(query_tokens, ((0, 0), (0, LP - L))).reshape(-1)
    dt = jnp.pad(doc_tokens, ((0, 0), (0, LP - L))).reshape(-1)
    pq = _pool_sc(qt, q_emb_table, q_boost_table, L, LP, B)
    pd = _pool_sc(dt, d_emb_table, d_boost_table, L, LP, B)
    oq = _mlp_tc(pq, q_W1, q_b1[None], q_g1[None], q_be1[None],
                 q_W2, q_b2[None], q_g2[None], q_be2[None], q_oW, q_ob[None])
    od = _mlp_tc(pd, d_W1, d_b1[None], d_g1[None], d_be1[None],
                 d_W2, d_b2[None], d_g2[None], d_be2[None], d_oW, d_ob[None])
    return jnp.stack([oq, od])


# speedup vs baseline: 5.9875x; 1.0137x over previous
import functools

import jax
import jax.numpy as jnp
import numpy as np
from jax import lax
from jax.experimental import pallas as pl
from jax.experimental.pallas import tpu as pltpu
from jax.experimental.pallas import tpu_sc as plsc

NC = 2
NS = 16
NW = NC * NS
LANE = 16

_SPLAT_DNUMS = lax.GatherDimensionNumbers(
    offset_dims=(), collapsed_slice_dims=(0,), start_index_map=(0,))


def _splat_lane(vec, lane):
    idx = jnp.full((LANE, 1), lane, jnp.int32)
    return lax.gather(vec, idx, _SPLAT_DNUMS, (1,),
                      mode=lax.GatherScatterMode.PROMISE_IN_BOUNDS)


def _pool_sc(tokens, emb, boost, L, LP, B):
    V, D = emb.shape
    rows_per_w = B // NW
    CH = 4
    nch = rows_per_w // CH
    TOK = rows_per_w * LP
    inv_l = 1.0 / L
    mesh = plsc.VectorSubcoreMesh(core_axis_name="c", subcore_axis_name="s")

    @functools.partial(
        pl.kernel,
        out_type=jax.ShapeDtypeStruct((B, D), jnp.float32),
        mesh=mesh,
        scratch_types=[
            pltpu.VMEM((TOK,), jnp.int32),
            pltpu.VMEM((rows_per_w * 128,), jnp.float32),
            pltpu.VMEM((CH * L, D), jnp.float32),
            pltpu.VMEM((CH * L, D), jnp.float32),
            pltpu.VMEM((rows_per_w, D), jnp.float32),
            pltpu.SemaphoreType.DMA,
            pltpu.SemaphoreType.DMA,
            pltpu.SemaphoreType.DMA,
            pltpu.SemaphoreType.DMA,
        ],
    )
    def k(t_hbm, e_hbm, b_hbm, out_hbm,
          idx_t, bst_t, rows_a, rows_b, acc_v,
          sem_ra, sem_rb, sem_ba, sem_bb):
        wid = lax.axis_index("s") * NC + lax.axis_index("c")
        base = wid * rows_per_w
        pltpu.sync_copy(t_hbm.at[pl.ds(base * LP, TOK)], idx_t)
        bufs = ((rows_a, sem_ra, sem_ba), (rows_b, sem_rb, sem_bb))

        def issue(ch, rbuf, sr, sb):
            for r in range(CH):
                isl = idx_t.at[pl.ds((ch * CH + r) * LP, L)]
                pltpu.async_copy(e_hbm.at[isl],
                                 rbuf.at[pl.ds(r * L, L)], sr)
                pltpu.async_copy(b_hbm.at[isl],
                                 bst_t.at[pl.ds((ch * CH + r) * 128, L)],
                                 sb)

        def wait(ch, rbuf, sr, sb):
            pltpu.make_async_copy(
                b_hbm.at[idx_t.at[pl.ds(ch * CH * LP, CH * L)]],
                bst_t.at[pl.ds(ch * CH * 128, CH * L)], sb).wait()
            pltpu.make_async_copy(
                e_hbm.at[idx_t.at[pl.ds(ch * CH * LP, CH * L)]],
                rbuf, sr).wait()

        def compute(ch, rbuf):
            def row_body(r, carry):
                grow = ch * CH + r
                wvecs = [bst_t[pl.ds(grow * 128 + g * LANE, LANE)] * inv_l
                         for g in range((L + LANE - 1) // LANE)]
                accs = [jnp.zeros((LANE,), jnp.float32)
                        for _ in range(D // LANE)]
                for l in range(L):
                    wsplat = _splat_lane(wvecs[l // LANE], l % LANE)
                    for c in range(D // LANE):
                        accs[c] = (accs[c]
                                   + rbuf[r * L + l, pl.ds(c * LANE, LANE)]
                                   * wsplat)
                for c in range(D // LANE):
                    acc_v[grow, pl.ds(c * LANE, LANE)] = accs[c]
                return carry

            lax.fori_loop(0, CH, row_body, 0)

        issue(0, *bufs[0])

        def body2(i, carry):
            ch0 = 2 * i
            issue(ch0 + 1, *bufs[1])
            wait(ch0, *bufs[0])
            compute(ch0, bufs[0][0])

            @pl.when(ch0 + 2 < nch)
            def _():
                issue(ch0 + 2, *bufs[0])

            wait(ch0 + 1, *bufs[1])
            compute(ch0 + 1, bufs[1][0])
            return carry

        lax.fori_loop(0, nch // 2, body2, 0)
        pltpu.sync_copy(acc_v, out_hbm.at[pl.ds(base, rows_per_w)])

    return k(tokens, emb, boost)


def _layer_norm(x, g, b):
    mu = jnp.mean(x, axis=-1, keepdims=True)
    var = jnp.mean((x - mu) ** 2, axis=-1, keepdims=True)
    return (x - mu) * lax.rsqrt(var + 1e-5) * g + b


def _mlp_tc(x, W1, b1, g1, be1, W2, b2, g2, be2, oW, ob):
    B, D = x.shape
    H1 = W1.shape[-1]
    H2 = W2.shape[-1]
    OUT = oW.shape[-1]
    BS = 1024

    def bdot(a, b):
        return jnp.dot(a.astype(jnp.bfloat16), b.astype(jnp.bfloat16),
                       preferred_element_type=jnp.float32)

    def body(x_ref, w1_ref, b1_ref, g1_ref, be1_ref, w2_ref, b2_ref, g2_ref,
             be2_ref, ow_ref, ob_ref, o_ref):
        h = bdot(x_ref[...], w1_ref[...])
        h = jnp.maximum(h + b1_ref[...], 0.0)
        h = _layer_norm(h, g1_ref[...], be1_ref[...])
        h = bdot(h, w2_ref[...])
        h = jnp.maximum(h + b2_ref[...], 0.0)
        h = _layer_norm(h, g2_ref[...], be2_ref[...])
        o_ref[...] = bdot(h, ow_ref[...]) + ob_ref[...]

    def wspec(shape):
        return pl.BlockSpec(shape, lambda i: (0,) * len(shape))

    return pl.pallas_call(
        body,
        grid=(B // BS,),
        in_specs=[
            pl.BlockSpec((BS, D), lambda i: (i, 0)),
            wspec((D, H1)), wspec((1, H1)), wspec((1, H1)), wspec((1, H1)),
            wspec((H1, H2)), wspec((1, H2)), wspec((1, H2)), wspec((1, H2)),
            wspec((H2, OUT)), wspec((1, OUT)),
        ],
        out_specs=pl.BlockSpec((BS, OUT), lambda i: (i, 0)),
        out_shape=jax.ShapeDtypeStruct((B, OUT), jnp.float32),
    )(x, W1, b1, g1, be1, W2, b2, g2, be2, oW, ob)


def kernel(query_tokens, doc_tokens,
           q_emb_table, q_boost_table, q_W1, q_b1, q_g1, q_be1,
           q_W2, q_b2, q_g2, q_be2, q_oW, q_ob,
           d_emb_table, d_boost_table, d_W1, d_b1, d_g1, d_be1,
           d_W2, d_b2, d_g2, d_be2, d_oW, d_ob):
    B, L = query_tokens.shape
    LP = ((L + 7) // 8) * 8
    qt = jnp.pad(query_tokens, ((0, 0), (0, LP - L))).reshape(-1)
    dt = jnp.pad(doc_tokens, ((0, 0), (0, LP - L))).reshape(-1)
    pq = _pool_sc(qt, q_emb_table, q_boost_table, L, LP, B)
    pd = _pool_sc(dt, d_emb_table, d_boost_table, L, LP, B)
    oq = _mlp_tc(pq, q_W1, q_b1[None], q_g1[None], q_be1[None],
                 q_W2, q_b2[None], q_g2[None], q_be2[None], q_oW, q_ob[None])
    od = _mlp_tc(pd, d_W1, d_b1[None], d_g1[None], d_be1[None],
                 d_W2, d_b2[None], d_g2[None], d_be2[None], d_oW, d_ob[None])
    return jnp.stack([oq, od])


# speedup vs baseline: 5.9883x; 1.0001x over previous
import functools

import jax
import jax.numpy as jnp
import numpy as np
from jax import lax
from jax.experimental import pallas as pl
from jax.experimental.pallas import tpu as pltpu
from jax.experimental.pallas import tpu_sc as plsc

NC = 2
NS = 16
NW = NC * NS
LANE = 16

_SPLAT_DNUMS = lax.GatherDimensionNumbers(
    offset_dims=(), collapsed_slice_dims=(0,), start_index_map=(0,))


def _splat_lane(vec, lane):
    idx = jnp.full((LANE, 1), lane, jnp.int32)
    return lax.gather(vec, idx, _SPLAT_DNUMS, (1,),
                      mode=lax.GatherScatterMode.PROMISE_IN_BOUNDS)


def _pool_sc(tokens, emb, boost, L, LP, B):
    V, D = emb.shape
    rows_per_w = B // NW
    CH = 4
    nch = rows_per_w // CH
    TOK = rows_per_w * LP
    inv_l = 1.0 / L
    mesh = plsc.VectorSubcoreMesh(core_axis_name="c", subcore_axis_name="s")

    @functools.partial(
        pl.kernel,
        out_type=jax.ShapeDtypeStruct((B, D), jnp.float32),
        mesh=mesh,
        scratch_types=[
            pltpu.VMEM((TOK,), jnp.int32),
            pltpu.VMEM((rows_per_w * 128,), jnp.float32),
            pltpu.VMEM((CH * L, D), jnp.float32),
            pltpu.VMEM((CH * L, D), jnp.float32),
            pltpu.VMEM((rows_per_w, D), jnp.float32),
            pltpu.SemaphoreType.DMA,
            pltpu.SemaphoreType.DMA,
            pltpu.SemaphoreType.DMA,
            pltpu.SemaphoreType.DMA,
        ],
    )
    def k(t_hbm, e_hbm, b_hbm, out_hbm,
          idx_t, bst_t, rows_a, rows_b, acc_v,
          sem_ra, sem_rb, sem_ba, sem_bb):
        wid = lax.axis_index("s") * NC + lax.axis_index("c")
        base = wid * rows_per_w
        pltpu.sync_copy(t_hbm.at[pl.ds(base * LP, TOK)], idx_t)
        bufs = ((rows_a, sem_ra, sem_ba), (rows_b, sem_rb, sem_bb))

        def issue(ch, rbuf, sr, sb):
            for r in range(CH):
                isl = idx_t.at[pl.ds((ch * CH + r) * LP, L)]
                pltpu.async_copy(e_hbm.at[isl],
                                 rbuf.at[pl.ds(r * L, L)], sr)
                pltpu.async_copy(b_hbm.at[isl],
                                 bst_t.at[pl.ds((ch * CH + r) * 128, L)],
                                 sb)

        def wait(ch, rbuf, sr, sb):
            pltpu.make_async_copy(
                b_hbm.at[idx_t.at[pl.ds(ch * CH * LP, CH * L)]],
                bst_t.at[pl.ds(ch * CH * 128, CH * L)], sb).wait()
            pltpu.make_async_copy(
                e_hbm.at[idx_t.at[pl.ds(ch * CH * LP, CH * L)]],
                rbuf, sr).wait()

        def compute(ch, rbuf):
            def row_body(r, carry):
                grow = ch * CH + r
                wvecs = [bst_t[pl.ds(grow * 128 + g * LANE, LANE)] * inv_l
                         for g in range((L + LANE - 1) // LANE)]
                accs = [jnp.zeros((LANE,), jnp.float32)
                        for _ in range(D // LANE)]
                for l in range(L):
                    wsplat = _splat_lane(wvecs[l // LANE], l % LANE)
                    for c in range(D // LANE):
                        accs[c] = (accs[c]
                                   + rbuf[r * L + l, pl.ds(c * LANE, LANE)]
                                   * wsplat)
                for c in range(D // LANE):
                    acc_v[grow, pl.ds(c * LANE, LANE)] = accs[c]
                return carry

            lax.fori_loop(0, CH, row_body, 0)

        issue(0, *bufs[0])

        def body2(i, carry):
            ch0 = 2 * i
            issue(ch0 + 1, *bufs[1])
            wait(ch0, *bufs[0])
            compute(ch0, bufs[0][0])

            @pl.when(ch0 + 2 < nch)
            def _():
                issue(ch0 + 2, *bufs[0])

            wait(ch0 + 1, *bufs[1])
            compute(ch0 + 1, bufs[1][0])
            return carry

        lax.fori_loop(0, nch // 2, body2, 0)
        pltpu.sync_copy(acc_v, out_hbm.at[pl.ds(base, rows_per_w)])

    return k(tokens, emb, boost)


def _layer_norm(x, g, b):
    mu = jnp.mean(x, axis=-1, keepdims=True)
    var = jnp.mean((x - mu) ** 2, axis=-1, keepdims=True)
    return (x - mu) * lax.rsqrt(var + 1e-5) * g + b


def _mlp_tc(x, W1, b1, g1, be1, W2, b2, g2, be2, oW, ob):
    B, D = x.shape
    H1 = W1.shape[-1]
    H2 = W2.shape[-1]
    OUT = oW.shape[-1]
    BS = 2048

    def bdot(a, b):
        return jnp.dot(a.astype(jnp.bfloat16), b.astype(jnp.bfloat16),
                       preferred_element_type=jnp.float32)

    def body(x_ref, w1_ref, b1_ref, g1_ref, be1_ref, w2_ref, b2_ref, g2_ref,
             be2_ref, ow_ref, ob_ref, o_ref):
        h = bdot(x_ref[...], w1_ref[...])
        h = jnp.maximum(h + b1_ref[...], 0.0)
        h = _layer_norm(h, g1_ref[...], be1_ref[...])
        h = bdot(h, w2_ref[...])
        h = jnp.maximum(h + b2_ref[...], 0.0)
        h = _layer_norm(h, g2_ref[...], be2_ref[...])
        o_ref[...] = bdot(h, ow_ref[...]) + ob_ref[...]

    def wspec(shape):
        return pl.BlockSpec(shape, lambda i: (0,) * len(shape))

    return pl.pallas_call(
        body,
        grid=(B // BS,),
        in_specs=[
            pl.BlockSpec((BS, D), lambda i: (i, 0)),
            wspec((D, H1)), wspec((1, H1)), wspec((1, H1)), wspec((1, H1)),
            wspec((H1, H2)), wspec((1, H2)), wspec((1, H2)), wspec((1, H2)),
            wspec((H2, OUT)), wspec((1, OUT)),
        ],
        out_specs=pl.BlockSpec((BS, OUT), lambda i: (i, 0)),
        out_shape=jax.ShapeDtypeStruct((B, OUT), jnp.float32),
    )(x, W1, b1, g1, be1, W2, b2, g2, be2, oW, ob)


def kernel(query_tokens, doc_tokens,
           q_emb_table, q_boost_table, q_W1, q_b1, q_g1, q_be1,
           q_W2, q_b2, q_g2, q_be2, q_oW, q_ob,
           d_emb_table, d_boost_table, d_W1, d_b1, d_g1, d_be1,
           d_W2, d_b2, d_g2, d_be2, d_oW, d_ob):
    B, L = query_tokens.shape
    LP = ((L + 7) // 8) * 8
    qt = jnp.pad(query_tokens, ((0, 0), (0, LP - L))).reshape(-1)
    dt = jnp.pad(doc_tokens, ((0, 0), (0, LP - L))).reshape(-1)
    pq = _pool_sc(qt, q_emb_table, q_boost_table, L, LP, B)
    pd = _pool_sc(dt, d_emb_table, d_boost_table, L, LP, B)
    oq = _mlp_tc(pq, q_W1, q_b1[None], q_g1[None], q_be1[None],
                 q_W2, q_b2[None], q_g2[None], q_be2[None], q_oW, q_ob[None])
    od = _mlp_tc(pd, d_W1, d_b1[None], d_g1[None], d_be1[None],
                 d_W2, d_b2[None], d_g2[None], d_be2[None], d_oW, d_ob[None])
    return jnp.stack([oq, od])


# speedup vs baseline: 5.9921x; 1.0006x over previous
import functools

import jax
import jax.numpy as jnp
from jax import lax
from jax.experimental import pallas as pl
from jax.experimental.pallas import tpu as pltpu
from jax.experimental.pallas import tpu_sc as plsc

NC = 2
NS = 16
NW = NC * NS
LANE = 16

_SPLAT_DNUMS = lax.GatherDimensionNumbers(
    offset_dims=(), collapsed_slice_dims=(0,), start_index_map=(0,))


def _splat_lane(vec, lane):
    idx = jnp.full((LANE, 1), lane, jnp.int32)
    return lax.gather(vec, idx, _SPLAT_DNUMS, (1,),
                      mode=lax.GatherScatterMode.PROMISE_IN_BOUNDS)


def _pool_sc(tokens, emb, boost, L, LP, B):
    V, D = emb.shape
    rows_per_w = B // NW
    CH = 4
    nch = rows_per_w // CH
    TOK = rows_per_w * LP
    inv_l = 1.0 / L
    mesh = plsc.VectorSubcoreMesh(core_axis_name="c", subcore_axis_name="s")

    @functools.partial(
        pl.kernel,
        out_type=jax.ShapeDtypeStruct((B, D), jnp.float32),
        mesh=mesh,
        scratch_types=[
            pltpu.VMEM((TOK,), jnp.int32),
            pltpu.VMEM((rows_per_w * 128,), jnp.float32),
            pltpu.VMEM((CH * L, D), jnp.float32),
            pltpu.VMEM((CH * L, D), jnp.float32),
            pltpu.VMEM((rows_per_w, D), jnp.float32),
            pltpu.SemaphoreType.DMA,
            pltpu.SemaphoreType.DMA,
            pltpu.SemaphoreType.DMA,
            pltpu.SemaphoreType.DMA,
        ],
    )
    def k(t_hbm, e_hbm, b_hbm, out_hbm,
          idx_t, bst_t, rows_a, rows_b, acc_v,
          sem_ra, sem_rb, sem_ba, sem_bb):
        wid = lax.axis_index("s") * NC + lax.axis_index("c")
        base = wid * rows_per_w
        pltpu.sync_copy(t_hbm.at[pl.ds(base * LP, TOK)], idx_t)
        bufs = ((rows_a, sem_ra, sem_ba), (rows_b, sem_rb, sem_bb))

        def issue(ch, rbuf, sr, sb):
            for r in range(CH):
                isl = idx_t.at[pl.ds((ch * CH + r) * LP, L)]
                pltpu.async_copy(e_hbm.at[isl],
                                 rbuf.at[pl.ds(r * L, L)], sr)
                pltpu.async_copy(b_hbm.at[isl],
                                 bst_t.at[pl.ds((ch * CH + r) * 128, L)],
                                 sb)

        def wait(ch, rbuf, sr, sb):
            pltpu.make_async_copy(
                b_hbm.at[idx_t.at[pl.ds(ch * CH * LP, CH * L)]],
                bst_t.at[pl.ds(ch * CH * 128, CH * L)], sb).wait()
            pltpu.make_async_copy(
                e_hbm.at[idx_t.at[pl.ds(ch * CH * LP, CH * L)]],
                rbuf, sr).wait()

        def compute(ch, rbuf):
            def row_body(r, carry):
                grow = ch * CH + r
                wvecs = [bst_t[pl.ds(grow * 128 + g * LANE, LANE)] * inv_l
                         for g in range((L + LANE - 1) // LANE)]
                accs = [jnp.zeros((LANE,), jnp.float32)
                        for _ in range(D // LANE)]
                for l in range(L):
                    wsplat = _splat_lane(wvecs[l // LANE], l % LANE)
                    for c in range(D // LANE):
                        accs[c] = (accs[c]
                                   + rbuf[r * L + l, pl.ds(c * LANE, LANE)]
                                   * wsplat)
                for c in range(D // LANE):
                    acc_v[grow, pl.ds(c * LANE, LANE)] = accs[c]
                return carry

            lax.fori_loop(0, CH, row_body, 0)

        issue(0, *bufs[0])

        def body2(i, carry):
            ch0 = 2 * i
            issue(ch0 + 1, *bufs[1])
            wait(ch0, *bufs[0])
            compute(ch0, bufs[0][0])

            @pl.when(ch0 + 2 < nch)
            def _():
                issue(ch0 + 2, *bufs[0])

            wait(ch0 + 1, *bufs[1])
            compute(ch0 + 1, bufs[1][0])
            return carry

        lax.fori_loop(0, nch // 2, body2, 0)
        pltpu.sync_copy(acc_v, out_hbm.at[pl.ds(base, rows_per_w)])

    return k(tokens, emb, boost)


def _layer_norm(x, g, b):
    mu = jnp.mean(x, axis=-1, keepdims=True)
    var = jnp.mean((x - mu) ** 2, axis=-1, keepdims=True)
    return (x - mu) * lax.rsqrt(var + 1e-5) * g + b


def _mlp_tc(x, W1, b1, g1, be1, W2, b2, g2, be2, oW, ob):
    B, D = x.shape
    H1 = W1.shape[-1]
    H2 = W2.shape[-1]
    OUT = oW.shape[-1]
    BS = 2048

    def bdot(a, b):
        return jnp.dot(a.astype(jnp.bfloat16), b.astype(jnp.bfloat16),
                       preferred_element_type=jnp.float32)

    def body(x_ref, w1_ref, b1_ref, g1_ref, be1_ref, w2_ref, b2_ref, g2_ref,
             be2_ref, ow_ref, ob_ref, o_ref):
        h = bdot(x_ref[...], w1_ref[...])
        h = jnp.maximum(h + b1_ref[...], 0.0)
        h = _layer_norm(h, g1_ref[...], be1_ref[...])
        h = bdot(h, w2_ref[...])
        h = jnp.maximum(h + b2_ref[...], 0.0)
        h = _layer_norm(h, g2_ref[...], be2_ref[...])
        o_ref[...] = bdot(h, ow_ref[...]) + ob_ref[...]

    def wspec(shape):
        return pl.BlockSpec(shape, lambda i: (0,) * len(shape))

    return pl.pallas_call(
        body,
        grid=(B // BS,),
        in_specs=[
            pl.BlockSpec((BS, D), lambda i: (i, 0)),
            wspec((D, H1)), wspec((1, H1)), wspec((1, H1)), wspec((1, H1)),
            wspec((H1, H2)), wspec((1, H2)), wspec((1, H2)), wspec((1, H2)),
            wspec((H2, OUT)), wspec((1, OUT)),
        ],
        out_specs=pl.BlockSpec((BS, OUT), lambda i: (i, 0)),
        out_shape=jax.ShapeDtypeStruct((B, OUT), jnp.float32),
    )(x, W1, b1, g1, be1, W2, b2, g2, be2, oW, ob)


def kernel(query_tokens, doc_tokens,
           q_emb_table, q_boost_table, q_W1, q_b1, q_g1, q_be1,
           q_W2, q_b2, q_g2, q_be2, q_oW, q_ob,
           d_emb_table, d_boost_table, d_W1, d_b1, d_g1, d_be1,
           d_W2, d_b2, d_g2, d_be2, d_oW, d_ob):
    B, L = query_tokens.shape
    LP = ((L + 7) // 8) * 8
    qt = jnp.pad(query_tokens, ((0, 0), (0, LP - L))).reshape(-1)
    dt = jnp.pad(doc_tokens, ((0, 0), (0, LP - L))).reshape(-1)
    pq = _pool_sc(qt, q_emb_table, q_boost_table, L, LP, B)
    pd = _pool_sc(dt, d_emb_table, d_boost_table, L, LP, B)
    oq = _mlp_tc(pq, q_W1, q_b1[None], q_g1[None], q_be1[None],
                 q_W2, q_b2[None], q_g2[None], q_be2[None], q_oW, q_ob[None])
    od = _mlp_tc(pd, d_W1, d_b1[None], d_g1[None], d_be1[None],
                 d_W2, d_b2[None], d_g2[None], d_be2[None], d_oW, d_ob[None])
    return jnp.stack([oq, od])
